# baseline (speedup 1.0000x reference)
import functools

import jax
import jax.numpy as jnp
from jax import lax
from jax.experimental import pallas as pl
from jax.experimental.pallas import tpu as pltpu
from jax.experimental.pallas import tpu_sc as plsc

N = 10000
D = 128
H = 128
NC = 2
NS = 16
NW = NC * NS
E = 320000
EPT = E // NW
CH = 112
PEPT = 10192
NCHUNK = PEPT // CH
NSEG = 7
SEGC = NCHUNK // NSEG
NP = 10240
RPT = NP // NS


def _sc_aggregate(table, src, dst, zeros):
    mesh = plsc.VectorSubcoreMesh(core_axis_name="c", subcore_axis_name="s")

    @functools.partial(
        pl.kernel,
        mesh=mesh,
        out_type=jax.ShapeDtypeStruct((NC, NP, H), jnp.float32),
        scratch_types=[
            pltpu.VMEM((SEGC, CH), jnp.int32),
            pltpu.VMEM((SEGC, CH), jnp.int32),
            pltpu.VMEM((CH, H), jnp.float32),
            pltpu.VMEM((CH, H), jnp.float32),
            pltpu.VMEM((CH, H), jnp.float32),
            pltpu.VMEM_SHARED((NP, H), jnp.float32),
            pltpu.SemaphoreType.DMA,
            pltpu.SemaphoreType.DMA,
            pltpu.SemaphoreType.DMA,
        ],
    )
    def k(table_hbm, src_hbm, dst_hbm, zeros_hbm, out_hbm,
          src_v, dst_v, rows_a, rows_b, rows_c, acc_sh,
          sem_a, sem_b, sem_c):
        c = lax.axis_index("c")
        s = lax.axis_index("s")
        wid = s * NC + c
        base = pl.multiple_of(s * RPT, 8)
        pltpu.sync_copy(zeros_hbm, acc_sh.at[pl.ds(base, RPT)])
        plsc.subcore_barrier()

        def gat(j, buf, sem):
            return pltpu.make_async_copy(table_hbm.at[src_v.at[j]], buf, sem)

        def sca(j, buf):
            pltpu.sync_copy(buf, acc_sh.at[dst_v.at[j]], add=True)

        for seg in range(NSEG):
            pltpu.sync_copy(src_hbm.at[wid, seg], src_v)
            pltpu.sync_copy(dst_hbm.at[wid, seg], dst_v)
            gat(0, rows_a, sem_a).start()
            gat(1, rows_b, sem_b).start()

            def body(i, carry):
                j = 3 * i
                gat(j + 2, rows_c, sem_c).start()
                gat(j, rows_a, sem_a).wait()
                sca(j, rows_a)
                gat(j + 3, rows_a, sem_a).start()
                gat(j + 1, rows_b, sem_b).wait()
                sca(j + 1, rows_b)
                gat(j + 4, rows_b, sem_b).start()
                gat(j + 2, rows_c, sem_c).wait()
                sca(j + 2, rows_c)
                return carry

            lax.fori_loop(0, (SEGC - 4) // 3, body, 0)
            j = 3 * ((SEGC - 4) // 3)
            gat(j + 2, rows_c, sem_c).start()
            gat(j, rows_a, sem_a).wait()
            sca(j, rows_a)
            gat(j + 3, rows_a, sem_a).start()
            gat(j + 1, rows_b, sem_b).wait()
            sca(j + 1, rows_b)
            gat(j + 2, rows_c, sem_c).wait()
            sca(j + 2, rows_c)
            gat(j + 3, rows_a, sem_a).wait()
            sca(j + 3, rows_a)
        plsc.subcore_barrier()
        pltpu.sync_copy(acc_sh.at[pl.ds(base, RPT)],
                        out_hbm.at[c, pl.ds(base, RPT)])

    return k(table, src, dst, zeros)


def _bn(x, g, b):
    m = jnp.mean(x, axis=0, keepdims=True)
    v = jnp.mean(jnp.square(x - m), axis=0, keepdims=True)
    return g * (x - m) * jax.lax.rsqrt(v + 1e-5) + b


def _mlp(c, W1, b1, g1, be1, W2, b2, g2, be2):
    h1 = jnp.dot(c, W1, preferred_element_type=jnp.float32) + b1
    h1 = jax.nn.relu(_bn(h1, g1, be1))
    h2 = jnp.dot(h1, W2, preferred_element_type=jnp.float32) + b2
    return _bn(h2, g2, be2)


def _proj_body(x_ref, W_ref, b_ref, o_ref):
    o_ref[...] = (jnp.dot(x_ref[...], W_ref[...],
                          preferred_element_type=jnp.float32) + b_ref[...])


def _layer_body(h_ref, agg_ref, W1_ref, b1_ref, g1_ref, be1_ref,
                W2_ref, b2_ref, g2_ref, be2_ref, scal_ref, o_ref):
    h = h_ref[...]
    combined = (scal_ref[0] * h + agg_ref[0, :N].astype(jnp.float32)
                + agg_ref[1, :N].astype(jnp.float32))
    out = h + _mlp(combined, W1_ref[...], b1_ref[...], g1_ref[...],
                   be1_ref[...], W2_ref[...], b2_ref[...], g2_ref[...],
                   be2_ref[...])
    a = scal_ref[1]
    o_ref[...] = jnp.maximum(out, 0.0) + a * jnp.minimum(out, 0.0)


def _final_body(h_ref, agg_ref, W1_ref, b1_ref, g1_ref, be1_ref,
                W2_ref, b2_ref, g2_ref, be2_ref,
                Wh1_ref, bh1_ref, Wh2_ref, bh2_ref, scal_ref, o_ref):
    h = h_ref[...]
    combined = (scal_ref[0] * h + agg_ref[0, :N].astype(jnp.float32)
                + agg_ref[1, :N].astype(jnp.float32))
    u = h + _mlp(combined, W1_ref[...], b1_ref[...], g1_ref[...],
                 be1_ref[...], W2_ref[...], b2_ref[...], g2_ref[...],
                 be2_ref[...])
    o = jax.nn.relu(jnp.dot(u, Wh1_ref[...],
                            preferred_element_type=jnp.float32) + bh1_ref[...])
    o_ref[...] = (jnp.dot(o, Wh2_ref[...],
                          preferred_element_type=jnp.float32) + bh2_ref[...])


def _tc_call(body, n_in, out_shape, smem_args=0):
    in_specs = [pl.BlockSpec(memory_space=pltpu.VMEM)] * n_in
    if smem_args:
        in_specs[-smem_args:] = [pl.BlockSpec(memory_space=pltpu.SMEM)] * smem_args
    if isinstance(out_shape, tuple):
        out_specs = tuple(pl.BlockSpec(memory_space=pltpu.VMEM)
                          for _ in out_shape)
    else:
        out_specs = pl.BlockSpec(memory_space=pltpu.VMEM)
    return pl.pallas_call(
        body,
        in_specs=in_specs,
        out_specs=out_specs,
        out_shape=out_shape,
    )


def kernel(x_user, x_question, x_answer, params,
           edge_user_asks_question, edge_user_answers_question,
           edge_question_contains_answer, edge_user_rates_answer,
           edge_user_similar_to_user):
    p = params
    mlp = p["mlps"]["user_similar_to_user"]
    f32 = jnp.float32

    pad = ((0, 0), (0, PEPT - EPT))
    src = jnp.pad(edge_user_similar_to_user[0].reshape(NW, EPT), pad,
                  constant_values=0).reshape(NW, NSEG, SEGC, CH)
    dst = jnp.pad(edge_user_similar_to_user[1].reshape(NW, EPT), pad,
                  constant_values=N).reshape(NW, NSEG, SEGC, CH)
    zeros = jnp.zeros((RPT, H), jnp.float32)

    scal = jnp.stack([1.0 + p["epsilon"], p["prelu"]]).astype(f32)

    def b2d(b):
        return b.reshape(1, H)

    h = _tc_call(
        _proj_body, 3, jax.ShapeDtypeStruct((N, H), f32))(
        x_user, p["proj"]["user"]["W"], b2d(p["proj"]["user"]["b"]))

    mlp_args = (mlp["lin1"]["W"], b2d(mlp["lin1"]["b"]), b2d(mlp["g1"]),
                b2d(mlp["be1"]), mlp["lin2"]["W"], b2d(mlp["lin2"]["b"]),
                b2d(mlp["g2"]), b2d(mlp["be2"]))

    agg = _sc_aggregate(h, src, dst, zeros)
    h = _tc_call(
        _layer_body, 11, jax.ShapeDtypeStruct((N, H), f32),
        smem_args=1)(h, agg, *mlp_args, scal)

    agg = _sc_aggregate(h, src, dst, zeros)
    out = _tc_call(_final_body, 15, jax.ShapeDtypeStruct((N, H), f32),
                   smem_args=1)(
        h, agg, *mlp_args,
        p["out"]["lin1"]["W"], b2d(p["out"]["lin1"]["b"]),
        p["out"]["lin2"]["W"], b2d(p["out"]["lin2"]["b"]), scal)
    return out


# speedup vs baseline: 2.6810x; 2.6810x over previous
import functools

import jax
import jax.numpy as jnp
from jax import lax
from jax.experimental import pallas as pl
from jax.experimental.pallas import tpu as pltpu
from jax.experimental.pallas import tpu_sc as plsc

N = 10000
D = 128
H = 128
NC = 2
NS = 16
NW = NC * NS
E = 320000
EPT = E // NW
CH = 80
PEPT = 10000
NCHUNK = PEPT // CH
NSEG = 5
SEGC = NCHUNK // NSEG
NP = 10240
RPT = NP // NS


def _sc_aggregate(table, src, dst, zeros):
    mesh = plsc.VectorSubcoreMesh(core_axis_name="c", subcore_axis_name="s")

    @functools.partial(
        pl.kernel,
        mesh=mesh,
        out_type=jax.ShapeDtypeStruct((NC, NP, H), jnp.float32),
        scratch_types=[
            pltpu.VMEM((SEGC, CH), jnp.int32),
            pltpu.VMEM((SEGC, CH), jnp.int32),
            pltpu.VMEM((CH, H), jnp.float32),
            pltpu.VMEM((CH, H), jnp.float32),
            pltpu.VMEM((CH, H), jnp.float32),
            pltpu.VMEM((CH, H), jnp.float32),
            pltpu.VMEM_SHARED((NP, H), jnp.float32),
            pltpu.SemaphoreType.DMA,
            pltpu.SemaphoreType.DMA,
            pltpu.SemaphoreType.DMA,
            pltpu.SemaphoreType.DMA,
            pltpu.SemaphoreType.DMA,
        ],
    )
    def k(table_hbm, src_hbm, dst_hbm, zeros_hbm, out_hbm,
          src_v, dst_v, rows_a, rows_b, rows_c, rows_d, acc_sh,
          sem_a, sem_b, sem_c, sem_d, sem_z):
        c = lax.axis_index("c")
        s = lax.axis_index("s")
        wid = s * NC + c
        base = pl.multiple_of(s * RPT, 8)

        def zc():
            return pltpu.make_async_copy(
                zeros_hbm, acc_sh.at[pl.ds(base, RPT)], sem_z)

        zc().start()

        def gat(j, buf, sem):
            return pltpu.make_async_copy(table_hbm.at[src_v.at[j]], buf, sem)

        def sca(j, buf):
            pltpu.sync_copy(buf, acc_sh.at[dst_v.at[j]], add=True)

        for seg in range(NSEG):
            pltpu.sync_copy(src_hbm.at[wid, seg], src_v)
            pltpu.sync_copy(dst_hbm.at[wid, seg], dst_v)
            gat(0, rows_a, sem_a).start()
            gat(1, rows_b, sem_b).start()
            gat(2, rows_c, sem_c).start()
            if seg == 0:
                zc().wait()
                plsc.subcore_barrier()

            def body(i, carry):
                j = 4 * i
                gat(j + 3, rows_d, sem_d).start()
                gat(j, rows_a, sem_a).wait()
                sca(j, rows_a)
                gat(j + 4, rows_a, sem_a).start()
                gat(j + 1, rows_b, sem_b).wait()
                sca(j + 1, rows_b)
                gat(j + 5, rows_b, sem_b).start()
                gat(j + 2, rows_c, sem_c).wait()
                sca(j + 2, rows_c)
                gat(j + 6, rows_c, sem_c).start()
                gat(j + 3, rows_d, sem_d).wait()
                sca(j + 3, rows_d)
                return carry

            lax.fori_loop(0, (SEGC - 5) // 4, body, 0)
            j = 4 * ((SEGC - 5) // 4)
            gat(j + 3, rows_d, sem_d).start()
            gat(j, rows_a, sem_a).wait()
            sca(j, rows_a)
            gat(j + 4, rows_a, sem_a).start()
            gat(j + 1, rows_b, sem_b).wait()
            sca(j + 1, rows_b)
            gat(j + 2, rows_c, sem_c).wait()
            sca(j + 2, rows_c)
            gat(j + 3, rows_d, sem_d).wait()
            sca(j + 3, rows_d)
            gat(j + 4, rows_a, sem_a).wait()
            sca(j + 4, rows_a)
        plsc.subcore_barrier()
        pltpu.sync_copy(acc_sh.at[pl.ds(base, RPT)],
                        out_hbm.at[c, pl.ds(base, RPT)])

    return k(table, src, dst, zeros)


def _bn(x, g, b):
    m = jnp.mean(x, axis=0, keepdims=True)
    v = jnp.mean(jnp.square(x - m), axis=0, keepdims=True)
    return g * (x - m) * jax.lax.rsqrt(v + 1e-5) + b


def _mlp(c, W1, b1, g1, be1, W2, b2, g2, be2):
    h1 = jnp.dot(c, W1, preferred_element_type=jnp.float32) + b1
    h1 = jax.nn.relu(_bn(h1, g1, be1))
    h2 = jnp.dot(h1, W2, preferred_element_type=jnp.float32) + b2
    return _bn(h2, g2, be2)


def _proj_body(x_ref, W_ref, b_ref, o_ref):
    o_ref[...] = (jnp.dot(x_ref[...], W_ref[...],
                          preferred_element_type=jnp.float32) + b_ref[...])


def _layer_body(h_ref, agg_ref, W1_ref, b1_ref, g1_ref, be1_ref,
                W2_ref, b2_ref, g2_ref, be2_ref, scal_ref, o_ref):
    h = h_ref[...]
    combined = (scal_ref[0] * h + agg_ref[0, :N].astype(jnp.float32)
                + agg_ref[1, :N].astype(jnp.float32))
    out = h + _mlp(combined, W1_ref[...], b1_ref[...], g1_ref[...],
                   be1_ref[...], W2_ref[...], b2_ref[...], g2_ref[...],
                   be2_ref[...])
    a = scal_ref[1]
    o_ref[...] = jnp.maximum(out, 0.0) + a * jnp.minimum(out, 0.0)


def _final_body(h_ref, agg_ref, W1_ref, b1_ref, g1_ref, be1_ref,
                W2_ref, b2_ref, g2_ref, be2_ref,
                Wh1_ref, bh1_ref, Wh2_ref, bh2_ref, scal_ref, o_ref):
    h = h_ref[...]
    combined = (scal_ref[0] * h + agg_ref[0, :N].astype(jnp.float32)
                + agg_ref[1, :N].astype(jnp.float32))
    u = h + _mlp(combined, W1_ref[...], b1_ref[...], g1_ref[...],
                 be1_ref[...], W2_ref[...], b2_ref[...], g2_ref[...],
                 be2_ref[...])
    o = jax.nn.relu(jnp.dot(u, Wh1_ref[...],
                            preferred_element_type=jnp.float32) + bh1_ref[...])
    o_ref[...] = (jnp.dot(o, Wh2_ref[...],
                          preferred_element_type=jnp.float32) + bh2_ref[...])


def _tc_call(body, n_in, out_shape, smem_args=0):
    in_specs = [pl.BlockSpec(memory_space=pltpu.VMEM)] * n_in
    if smem_args:
        in_specs[-smem_args:] = [pl.BlockSpec(memory_space=pltpu.SMEM)] * smem_args
    if isinstance(out_shape, tuple):
        out_specs = tuple(pl.BlockSpec(memory_space=pltpu.VMEM)
                          for _ in out_shape)
    else:
        out_specs = pl.BlockSpec(memory_space=pltpu.VMEM)
    return pl.pallas_call(
        body,
        in_specs=in_specs,
        out_specs=out_specs,
        out_shape=out_shape,
    )


def kernel(x_user, x_question, x_answer, params,
           edge_user_asks_question, edge_user_answers_question,
           edge_question_contains_answer, edge_user_rates_answer,
           edge_user_similar_to_user):
    p = params
    mlp = p["mlps"]["user_similar_to_user"]
    f32 = jnp.float32

    pad = ((0, 0), (0, PEPT - EPT))
    src = jnp.pad(edge_user_similar_to_user[0].reshape(NW, EPT), pad,
                  constant_values=0).reshape(NW, NSEG, SEGC, CH)
    dst = jnp.pad(edge_user_similar_to_user[1].reshape(NW, EPT), pad,
                  constant_values=N).reshape(NW, NSEG, SEGC, CH)
    zeros = jnp.zeros((RPT, H), jnp.float32)

    scal = jnp.stack([1.0 + p["epsilon"], p["prelu"]]).astype(f32)

    def b2d(b):
        return b.reshape(1, H)

    h = _tc_call(
        _proj_body, 3, jax.ShapeDtypeStruct((N, H), f32))(
        x_user, p["proj"]["user"]["W"], b2d(p["proj"]["user"]["b"]))

    mlp_args = (mlp["lin1"]["W"], b2d(mlp["lin1"]["b"]), b2d(mlp["g1"]),
                b2d(mlp["be1"]), mlp["lin2"]["W"], b2d(mlp["lin2"]["b"]),
                b2d(mlp["g2"]), b2d(mlp["be2"]))

    agg = _sc_aggregate(h, src, dst, zeros)
    h = _tc_call(
        _layer_body, 11, jax.ShapeDtypeStruct((N, H), f32),
        smem_args=1)(h, agg, *mlp_args, scal)

    agg = _sc_aggregate(h, src, dst, zeros)
    out = _tc_call(_final_body, 15, jax.ShapeDtypeStruct((N, H), f32),
                   smem_args=1)(
        h, agg, *mlp_args,
        p["out"]["lin1"]["W"], b2d(p["out"]["lin1"]["b"]),
        p["out"]["lin2"]["W"], b2d(p["out"]["lin2"]["b"]), scal)
    return out
